# baseline (speedup 1.0000x reference)
import functools

import jax
import jax.numpy as jnp
from jax import lax
from jax.experimental import pallas as pl
from jax.experimental.pallas import tpu as pltpu, tpu_sc as plsc

VOCAB = 1000000
D = 32
MAX_LEN = 200
BATCH = 16384
SEQ = 200
TOTAL = BATCH * SEQ

NC, NS = 2, 16
NW = NC * NS
G = 128
C = 1024
NG = C // G
PER_W = TOTAL // NW
CHUNKS = PER_W // C
GU_PER_W = PER_W // G


def _emb_kernel(tok_hbm, pos_hbm, item_hbm, ptab_hbm, out_hbm,
                idx_t0, idx_t1, idx_p0, idx_p1, rows0, rows1, ptab_v,
                gsem0, gsem1, wsem0, wsem1):
    wid = lax.axis_index("c") * NS + lax.axis_index("s")
    gu_base = wid * GU_PER_W
    el_base = wid * PER_W

    idx_t = (idx_t0, idx_t1)
    idx_p = (idx_p0, idx_p1)
    rows = (rows0, rows1)
    gsem = (gsem0, gsem1)
    wsem = (wsem0, wsem1)

    pltpu.sync_copy(ptab_hbm, ptab_v)

    def stage_and_fire(i, b):
        pltpu.sync_copy(tok_hbm.at[pl.ds(gu_base + i * NG, NG)], idx_t[b])
        pltpu.sync_copy(pos_hbm.at[pl.ds(el_base + i * C, C)], idx_p[b])
        for j in range(NG):
            pltpu.async_copy(item_hbm.at[idx_t[b].at[j]],
                             rows[b].at[pl.ds(j * G, G)],
                             gsem[b])

    def wait_gathers(b):
        for j in range(NG):
            pltpu.make_async_copy(item_hbm.at[idx_t[b].at[j]],
                                  rows[b].at[pl.ds(j * G, G)],
                                  gsem[b]).wait()

    def wb_descr(i, b):
        return pltpu.make_async_copy(
            rows[b],
            out_hbm.at[pl.ds(el_base + i * C, C)], wsem[b])

    iota16 = lax.iota(jnp.int32, 16)
    eofs0 = iota16 * D

    def add_pos(b):
        rbuf = rows[b]
        pbuf = idx_p[b]

        def add_body(g, carry):
            pf = pbuf[pl.ds(g * 16, 16)] * D
            eids = iota16 + g * 16
            for d in range(D):
                dvec = jnp.full((16,), d, jnp.int32)
                pv = plsc.load_gather(ptab_v, [pf + d])
                plsc.addupdate_scatter(rbuf, [eids, dvec], pv)
            return carry

        lax.fori_loop(0, C // 16, add_body, 0)

    def step(i, b):
        wait_gathers(b)
        add_pos(b)
        wb_descr(i, b).start()
        nb = 1 - b

        @pl.when(i + 1 < CHUNKS)
        def _prefetch():
            stage_and_fire(i + 1, nb)

        @pl.when(i >= 1)
        def _drain_prev_wb():
            wb_descr(i - 1, nb).wait()

    stage_and_fire(0, 0)

    def pair_body(ii, carry):
        step(2 * ii, 0)
        step(2 * ii + 1, 1)
        return carry

    lax.fori_loop(0, CHUNKS // 2, pair_body, 0)
    wb_descr(CHUNKS - 1, (CHUNKS - 1) % 2).wait()


@jax.jit
def kernel(tokens, positions, item_table, pos_table):
    tok2d = tokens.reshape(TOTAL // G, G)
    pos1d = positions.reshape(TOTAL)

    mesh = plsc.VectorSubcoreMesh(core_axis_name="c", subcore_axis_name="s")
    run = functools.partial(
        pl.kernel,
        out_type=jax.ShapeDtypeStruct((TOTAL, D), jnp.float32),
        mesh=mesh,
        scratch_types=[
            pltpu.VMEM((NG, G), jnp.int32),
            pltpu.VMEM((NG, G), jnp.int32),
            pltpu.VMEM((C,), jnp.int32),
            pltpu.VMEM((C,), jnp.int32),
            pltpu.VMEM((C, D), jnp.float32),
            pltpu.VMEM((C, D), jnp.float32),
            pltpu.VMEM((MAX_LEN * D,), jnp.float32),
            pltpu.SemaphoreType.DMA,
            pltpu.SemaphoreType.DMA,
            pltpu.SemaphoreType.DMA,
            pltpu.SemaphoreType.DMA,
        ],
        compiler_params=pltpu.CompilerParams(use_tc_tiling_on_sc=False,
                                             needs_layout_passes=False),
    )(_emb_kernel)
    out_flat = run(tok2d, pos1d, item_table, pos_table.reshape(MAX_LEN * D))
    return out_flat.reshape(BATCH, SEQ, D)


# speedup vs baseline: 1.1378x; 1.1378x over previous
import functools

import jax
import jax.numpy as jnp
from jax import lax
from jax.experimental import pallas as pl
from jax.experimental.pallas import tpu as pltpu, tpu_sc as plsc

VOCAB = 1000000
D = 32
MAX_LEN = 200
BATCH = 16384
SEQ = 200
TOTAL = BATCH * SEQ

NC, NS = 2, 16
NW = NC * NS
G = 128
C = 1024
NG = C // G
PER_W = TOTAL // NW
CHUNKS = PER_W // C
GU_PER_W = PER_W // G


def _emb_kernel(tok_hbm, pos_hbm, item_hbm, ptab_hbm, out_hbm,
                idx_t0, idx_t1, idx_p0, idx_p1, rows0, rows1, ptab_v,
                gsem0, gsem1, wsem0, wsem1):
    wid = lax.axis_index("c") * NS + lax.axis_index("s")
    gu_base = wid * GU_PER_W
    el_base = wid * PER_W

    idx_t = (idx_t0, idx_t1)
    idx_p = (idx_p0, idx_p1)
    rows = (rows0, rows1)
    gsem = (gsem0, gsem1)
    wsem = (wsem0, wsem1)

    pltpu.sync_copy(ptab_hbm, ptab_v)

    def stage_and_fire(i, b):
        pltpu.sync_copy(tok_hbm.at[pl.ds(gu_base + i * NG, NG)], idx_t[b])
        pltpu.sync_copy(pos_hbm.at[pl.ds(el_base + i * C, C)], idx_p[b])
        for j in range(NG):
            pltpu.async_copy(item_hbm.at[idx_t[b].at[j]],
                             rows[b].at[pl.ds(j * G, G)],
                             gsem[b])

    def wait_gathers(b):
        for j in range(NG):
            pltpu.make_async_copy(item_hbm.at[idx_t[b].at[j]],
                                  rows[b].at[pl.ds(j * G, G)],
                                  gsem[b]).wait()

    def wb_descr(i, b):
        return pltpu.make_async_copy(
            rows[b],
            out_hbm.at[pl.ds(el_base + i * C, C)], wsem[b])

    iota16 = lax.iota(jnp.int32, 16)
    eofs0 = iota16 * D

    def add_pos(b):
        rbuf = rows[b]
        pbuf = idx_p[b]

        @plsc.parallel_loop(0, C // 16, unroll=2)
        def add_body(g):
            pf = pbuf[pl.ds(g * 16, 16)] * D
            eids = iota16 + g * 16
            for d in range(D):
                dvec = jnp.full((16,), d, jnp.int32)
                pv = plsc.load_gather(ptab_v, [pf + d])
                plsc.addupdate_scatter(rbuf, [eids, dvec], pv)

    def step(i, b):
        wait_gathers(b)
        add_pos(b)
        wb_descr(i, b).start()
        nb = 1 - b

        @pl.when(i + 1 < CHUNKS)
        def _prefetch():
            stage_and_fire(i + 1, nb)

        @pl.when(i >= 1)
        def _drain_prev_wb():
            wb_descr(i - 1, nb).wait()

    stage_and_fire(0, 0)

    def pair_body(ii, carry):
        step(2 * ii, 0)
        step(2 * ii + 1, 1)
        return carry

    lax.fori_loop(0, CHUNKS // 2, pair_body, 0)
    wb_descr(CHUNKS - 1, (CHUNKS - 1) % 2).wait()


@jax.jit
def kernel(tokens, positions, item_table, pos_table):
    tok2d = tokens.reshape(TOTAL // G, G)
    pos1d = positions.reshape(TOTAL)

    mesh = plsc.VectorSubcoreMesh(core_axis_name="c", subcore_axis_name="s")
    run = functools.partial(
        pl.kernel,
        out_type=jax.ShapeDtypeStruct((TOTAL, D), jnp.float32),
        mesh=mesh,
        scratch_types=[
            pltpu.VMEM((NG, G), jnp.int32),
            pltpu.VMEM((NG, G), jnp.int32),
            pltpu.VMEM((C,), jnp.int32),
            pltpu.VMEM((C,), jnp.int32),
            pltpu.VMEM((C, D), jnp.float32),
            pltpu.VMEM((C, D), jnp.float32),
            pltpu.VMEM((MAX_LEN * D,), jnp.float32),
            pltpu.SemaphoreType.DMA,
            pltpu.SemaphoreType.DMA,
            pltpu.SemaphoreType.DMA,
            pltpu.SemaphoreType.DMA,
        ],
        compiler_params=pltpu.CompilerParams(use_tc_tiling_on_sc=False,
                                             needs_layout_passes=False),
    )(_emb_kernel)
    out_flat = run(tok2d, pos1d, item_table, pos_table.reshape(MAX_LEN * D))
    return out_flat.reshape(BATCH, SEQ, D)


# speedup vs baseline: 2.0426x; 1.7952x over previous
import functools

import jax
import jax.numpy as jnp
from jax import lax
from jax.experimental import pallas as pl
from jax.experimental.pallas import tpu as pltpu, tpu_sc as plsc

VOCAB = 1000000
D = 32
MAX_LEN = 200
BATCH = 16384
SEQ = 200
TOTAL = BATCH * SEQ

NC, NS = 2, 16
NW = NC * NS
G = 128
C = 1024
NG = C // G
PER_W = TOTAL // NW
CHUNKS = PER_W // C
GU_PER_W = PER_W // G


def _emb_kernel(tok_hbm, pos_hbm, item_hbm, ptab_hbm, out_hbm,
                idx_t0, idx_t1, idx_p0, idx_p1, rows0, rows1, ptab_v,
                gsem0, gsem1, wsem0, wsem1):
    wid = lax.axis_index("c") * NS + lax.axis_index("s")
    gu_base = wid * GU_PER_W
    el_base = wid * PER_W

    idx_t = (idx_t0, idx_t1)
    idx_p = (idx_p0, idx_p1)
    rows = (rows0, rows1)
    gsem = (gsem0, gsem1)
    wsem = (wsem0, wsem1)

    pltpu.sync_copy(ptab_hbm, ptab_v)

    def stage_and_fire(i, b):
        pltpu.sync_copy(tok_hbm.at[pl.ds(gu_base + i * NG, NG)], idx_t[b])
        pltpu.sync_copy(pos_hbm.at[pl.ds(el_base + i * C, C)], idx_p[b])
        for j in range(NG):
            pltpu.async_copy(item_hbm.at[idx_t[b].at[j]],
                             rows[b].at[pl.ds(j * G, G)],
                             gsem[b])

    def wait_gathers(b):
        for j in range(NG):
            pltpu.make_async_copy(item_hbm.at[idx_t[b].at[j]],
                                  rows[b].at[pl.ds(j * G, G)],
                                  gsem[b]).wait()

    def wb_descr(i, b):
        return pltpu.make_async_copy(
            rows[b],
            out_hbm.at[pl.ds(el_base + i * C, C)], wsem[b])

    iota16 = lax.iota(jnp.int32, 16)
    eofs0 = iota16 * D

    def add_pos(b):
        rbuf = rows[b]
        pbuf = idx_p[b]

        def add_body(g, carry):
            p16 = pbuf[pl.ds(g * 16, 16)] * D
            for k in range(16):
                e = g * 16 + k
                bofs = p16[k]
                for d0 in (0, 16):
                    pv = ptab_v[pl.ds(bofs + d0, 16)]
                    plsc.addupdate(rbuf.at[e, pl.ds(d0, 16)], pv)
            return carry

        lax.fori_loop(0, C // 16, add_body, 0)

    def step(i, b):
        wait_gathers(b)
        add_pos(b)
        wb_descr(i, b).start()
        nb = 1 - b

        @pl.when(i + 1 < CHUNKS)
        def _prefetch():
            stage_and_fire(i + 1, nb)

        @pl.when(i >= 1)
        def _drain_prev_wb():
            wb_descr(i - 1, nb).wait()

    stage_and_fire(0, 0)

    def pair_body(ii, carry):
        step(2 * ii, 0)
        step(2 * ii + 1, 1)
        return carry

    lax.fori_loop(0, CHUNKS // 2, pair_body, 0)
    wb_descr(CHUNKS - 1, (CHUNKS - 1) % 2).wait()


@jax.jit
def kernel(tokens, positions, item_table, pos_table):
    tok2d = tokens.reshape(TOTAL // G, G)
    pos1d = positions.reshape(TOTAL)

    mesh = plsc.VectorSubcoreMesh(core_axis_name="c", subcore_axis_name="s")
    run = functools.partial(
        pl.kernel,
        out_type=jax.ShapeDtypeStruct((TOTAL, D), jnp.float32),
        mesh=mesh,
        scratch_types=[
            pltpu.VMEM((NG, G), jnp.int32),
            pltpu.VMEM((NG, G), jnp.int32),
            pltpu.VMEM((C,), jnp.int32),
            pltpu.VMEM((C,), jnp.int32),
            pltpu.VMEM((C, D), jnp.float32),
            pltpu.VMEM((C, D), jnp.float32),
            pltpu.VMEM((MAX_LEN * D,), jnp.float32),
            pltpu.SemaphoreType.DMA,
            pltpu.SemaphoreType.DMA,
            pltpu.SemaphoreType.DMA,
            pltpu.SemaphoreType.DMA,
        ],
        compiler_params=pltpu.CompilerParams(use_tc_tiling_on_sc=False),
    )(_emb_kernel)
    out_flat = run(tok2d, pos1d, item_table, pos_table.reshape(MAX_LEN * D))
    return out_flat.reshape(BATCH, SEQ, D)
